# baseline (speedup 1.0000x reference)
import functools

import jax
import jax.numpy as jnp
from jax import lax
from jax.experimental import pallas as pl
from jax.experimental.pallas import tpu as pltpu
from jax.experimental.pallas import tpu_sc as plsc

_HIGH = jax.lax.Precision.HIGHEST

N = 2048
D = 1024
L = 8
TILE = 128
PMAX = 5120
NT = PMAX // TILE
NW = 32
RPW = PMAX // NW
TPW = N // NW


def _top2(z):
    idx = jax.lax.broadcasted_iota(jnp.int32, z.shape, 1)
    v1 = jnp.max(z, axis=1, keepdims=True)
    i1 = jnp.min(jnp.where(z == v1, idx, L), axis=1, keepdims=True)
    m1 = idx == i1
    z2 = jnp.where(m1, -jnp.inf, z)
    v2 = jnp.max(z2, axis=1, keepdims=True)
    i2 = jnp.min(jnp.where(z2 == v2, idx, L), axis=1, keepdims=True)
    m2 = idx == i2
    a = jnp.exp((v2 - v1) / (1.0 + 1e-8))
    w1 = 1.0 / (1.0 + a)
    w2 = a / (1.0 + a)
    return m1, m2, w1, w2


def _sort_positions(mask_f, tri, triu8):
    base = jnp.zeros((1, L), jnp.float32)
    ranks = []
    for i in range(N // TILE):
        chunk = mask_f[i * TILE:(i + 1) * TILE]
        local = jax.lax.dot_general(tri, chunk, (((1,), (0,)), ((), ())),
                                    precision=_HIGH,
                                    preferred_element_type=jnp.float32)
        ranks.append(local + base)
        base = base + jnp.sum(chunk, axis=0, keepdims=True)
    rank = jnp.concatenate(ranks, axis=0)
    cap = jnp.ceil(base / TILE) * TILE
    off = jax.lax.dot_general(cap, triu8, (((1,), (0,)), ((), ())),
                              precision=_HIGH,
                              preferred_element_type=jnp.float32)
    ntiles = (off[0, L - 1] + cap[0, L - 1]) / TILE
    return off + rank, off, ntiles


def _expert_per_tile(off, ntiles):
    tv = jax.lax.broadcasted_iota(jnp.int32, (1, 128), 1).astype(
        jnp.float32) * TILE
    acc = jnp.zeros((1, 128), jnp.float32)
    for l in range(L):
        acc = acc + jnp.where(tv >= off[0, l], 1.0, 0.0)
    et = acc - 1.0
    li = jax.lax.broadcasted_iota(jnp.int32, (1, 128), 1)
    et = jnp.where(li == 120, ntiles, et)
    return et.astype(jnp.int32)


def _route_body(x_ref, pw_ref, u_ref, b2_ref, tri_ref, tri8_ref,
                xb_ref, bias_ref, mi_ref, mf_ref, ets_ref):
    xt = x_ref[...]
    xb = xt.astype(jnp.bfloat16)
    xb_ref[...] = xb
    addr = jax.lax.dot_general(xb, pw_ref[...], (((1,), (0,)), ((), ())),
                               preferred_element_type=jnp.float32)
    zz = jax.lax.dot_general(addr.astype(jnp.bfloat16), u_ref[...],
                             (((1,), (0,)), ((), ())),
                             preferred_element_type=jnp.float32)
    tri = tri_ref[...]
    tri8 = tri8_ref[...]

    m1a, m1b, w1a, w1b = _top2(zz[:, 0:L])
    m2a, m2b, w2a, w2b = _top2(zz[:, L:2 * L])
    m3a, m3b, w3a, w3b = _top2(zz[:, 2 * L:3 * L])

    mask1 = jnp.where(m1a | m1b, 1.0, 0.0)
    mask2 = jnp.where(m2a | m2b, 1.0, 0.0)
    pos1, off1, nt1 = _sort_positions(mask1, tri, tri8)
    pos2, off2, nt2 = _sort_positions(mask2, tri, tri8)

    posA = jnp.sum(jnp.where(m1a, pos1, 0.0), axis=1, keepdims=True)
    posB = jnp.sum(jnp.where(m1b, pos1, 0.0), axis=1, keepdims=True)
    pos2A = jnp.sum(jnp.where(m2a, pos2, 0.0), axis=1, keepdims=True)
    pos2B = jnp.sum(jnp.where(m2b, pos2, 0.0), axis=1, keepdims=True)
    toks = jax.lax.broadcasted_iota(jnp.int32, (N, 1), 0).astype(jnp.float32)
    zc = jnp.zeros((N, 3), jnp.float32)
    mi = jnp.concatenate([posA, posB, pos2A, pos2B, toks, zc],
                         axis=1).astype(jnp.int32)
    mi_ref[...] = jax.lax.transpose(mi, (1, 0))
    mf = jnp.concatenate([w1a, w1b, w2a, w2b], axis=1)
    mf_ref[...] = jax.lax.transpose(mf, (1, 0))

    ets_ref[...] = jnp.concatenate(
        [_expert_per_tile(off1, nt1), _expert_per_tile(off2, nt2)], axis=0)

    c3 = jnp.where(m3a, w3a, 0.0) + jnp.where(m3b, w3b, 0.0)
    bias = jax.lax.dot_general(c3.astype(jnp.bfloat16), b2_ref[...],
                               (((1,), (0,)), ((), ())),
                               preferred_element_type=jnp.float32)
    bias_ref[...] = bias.astype(jnp.bfloat16)


def _route(x_flat, pwb, ub, b2b):
    ii = jax.lax.broadcasted_iota(jnp.int32, (TILE, TILE), 0)
    jj = jax.lax.broadcasted_iota(jnp.int32, (TILE, TILE), 1)
    tri = jnp.where(ii > jj, 1.0, 0.0).astype(jnp.float32)
    i8 = jax.lax.broadcasted_iota(jnp.int32, (L, L), 0)
    j8 = jax.lax.broadcasted_iota(jnp.int32, (L, L), 1)
    tri8 = jnp.where(i8 < j8, 1.0, 0.0).astype(jnp.float32)
    return pl.pallas_call(
        _route_body,
        grid=(1,),
        in_specs=[pl.BlockSpec(s, lambda i: tuple(0 for _ in s))
                  for s in ((N, D), pwb.shape, ub.shape, b2b.shape,
                            (TILE, TILE), (L, L))],
        out_specs=[pl.BlockSpec(s, lambda i: tuple(0 for _ in s))
                   for s in ((N, D), (N, D), (8, N), (4, N), (2, 128))],
        out_shape=[
            jax.ShapeDtypeStruct((N, D), jnp.bfloat16),
            jax.ShapeDtypeStruct((N, D), jnp.bfloat16),
            jax.ShapeDtypeStruct((8, N), jnp.int32),
            jax.ShapeDtypeStruct((4, N), jnp.float32),
            jax.ShapeDtypeStruct((2, 128), jnp.int32),
        ],
    )(x_flat, pwb, ub, b2b, tri, tri8)


def _sc_scatter_body(mi, mf, tok_o, g1_o, g2_o, ia_o, ib_o,
                     a, b, a2, b2v, ga, gb, g2a, g2b, ids, sem):
    wid = lax.axis_index("s") * 2 + lax.axis_index("c")
    t0 = wid * TPW
    pltpu.sync_copy(mi.at[0, pl.ds(t0, TPW)], a)
    pltpu.sync_copy(mi.at[1, pl.ds(t0, TPW)], b)
    pltpu.sync_copy(mi.at[2, pl.ds(t0, TPW)], a2)
    pltpu.sync_copy(mi.at[3, pl.ds(t0, TPW)], b2v)
    pltpu.sync_copy(mi.at[4, pl.ds(t0, TPW)], ids)
    pltpu.sync_copy(mf.at[0, pl.ds(t0, TPW)], ga)
    pltpu.sync_copy(mf.at[1, pl.ds(t0, TPW)], gb)
    pltpu.sync_copy(mf.at[2, pl.ds(t0, TPW)], g2a)
    pltpu.sync_copy(mf.at[3, pl.ds(t0, TPW)], g2b)
    pltpu.async_copy(ids, tok_o.at[a], sem).wait()
    pltpu.async_copy(ids, tok_o.at[b], sem).wait()
    pltpu.async_copy(ga, g1_o.at[a], sem).wait()
    pltpu.async_copy(gb, g1_o.at[b], sem).wait()
    pltpu.async_copy(g2a, g2_o.at[a2], sem).wait()
    pltpu.async_copy(g2b, g2_o.at[b2v], sem).wait()
    pltpu.async_copy(a, ia_o.at[a2], sem).wait()
    pltpu.async_copy(a, ia_o.at[b2v], sem).wait()
    pltpu.async_copy(b, ib_o.at[a2], sem).wait()
    pltpu.async_copy(b, ib_o.at[b2v], sem).wait()


def _sc_scatter(mi, mf):
    mesh = plsc.VectorSubcoreMesh(core_axis_name="c", subcore_axis_name="s")
    fn = pl.kernel(
        _sc_scatter_body,
        out_type=(
            jax.ShapeDtypeStruct((PMAX,), jnp.int32),
            jax.ShapeDtypeStruct((PMAX,), jnp.float32),
            jax.ShapeDtypeStruct((PMAX,), jnp.float32),
            jax.ShapeDtypeStruct((PMAX,), jnp.int32),
            jax.ShapeDtypeStruct((PMAX,), jnp.int32),
        ),
        mesh=mesh,
        scratch_types=[
            pltpu.VMEM((TPW,), jnp.int32), pltpu.VMEM((TPW,), jnp.int32),
            pltpu.VMEM((TPW,), jnp.int32), pltpu.VMEM((TPW,), jnp.int32),
            pltpu.VMEM((TPW,), jnp.float32), pltpu.VMEM((TPW,), jnp.float32),
            pltpu.VMEM((TPW,), jnp.float32), pltpu.VMEM((TPW,), jnp.float32),
            pltpu.VMEM((TPW,), jnp.int32),
            pltpu.SemaphoreType.DMA,
        ],
    )
    return fn(mi, mf)


def _clamp_idx(src_ref, dst_ref, n, bound):
    for c in range(n // 16):
        v = src_ref[pl.ds(c * 16, 16)]
        dst_ref[pl.ds(c * 16, 16)] = jnp.minimum(jnp.maximum(v, 0), bound - 1)


def _sc_gatherx_body(xsrc, tok, xs_o, idxr, idxc, rows, sem):
    wid = lax.axis_index("s") * 2 + lax.axis_index("c")
    lo = wid * RPW
    half = RPW // 2
    for c in range(2):
        pltpu.sync_copy(tok.at[pl.ds(lo + c * half, half)], idxr)
        _clamp_idx(idxr, idxc, half, N)
        pltpu.async_copy(xsrc.at[idxc], rows, sem).wait()
        pltpu.sync_copy(rows, xs_o.at[pl.ds(lo + c * half, half)])


def _sc_gatherx(x_flat, tok):
    mesh = plsc.VectorSubcoreMesh(core_axis_name="c", subcore_axis_name="s")
    fn = pl.kernel(
        _sc_gatherx_body,
        out_type=jax.ShapeDtypeStruct((PMAX, D), jnp.float32),
        mesh=mesh,
        scratch_types=[
            pltpu.VMEM((RPW // 2,), jnp.int32),
            pltpu.VMEM((RPW // 2,), jnp.int32),
            pltpu.VMEM((RPW // 2, D), jnp.float32),
            pltpu.SemaphoreType.DMA,
        ],
    )
    return fn(x_flat, tok)


def _gemm1_body(ets_ref, xs_ref, w_ref, g_ref, o_ref):
    t = pl.program_id(0)

    @pl.when(t < ets_ref[0, 120])
    def _():
        acc = jax.lax.dot_general(xs_ref[...].astype(jnp.bfloat16), w_ref[0],
                                  (((1,), (1,)), ((), ())),
                                  preferred_element_type=jnp.float32)
        gcol = jax.lax.transpose(g_ref[0], (1, 0))
        o_ref[...] = acc * gcol


def _gemm1(ets, xs, w1b, g1s):
    g3 = g1s.reshape(NT, 1, TILE)
    grid_spec = pltpu.PrefetchScalarGridSpec(
        num_scalar_prefetch=1,
        grid=(NT,),
        in_specs=[
            pl.BlockSpec((TILE, D), lambda i, ets: (i, 0)),
            pl.BlockSpec((1, D, D), lambda i, ets: (ets[0, i], 0, 0)),
            pl.BlockSpec((1, 1, TILE), lambda i, ets: (i, 0, 0)),
        ],
        out_specs=pl.BlockSpec((TILE, D), lambda i, ets: (i, 0)),
    )
    return pl.pallas_call(
        _gemm1_body, grid_spec=grid_spec,
        out_shape=jax.ShapeDtypeStruct((PMAX, D), jnp.float32),
    )(ets, xs, w1b, g3)


def _gemm2_body(ets_ref, ra_ref, rb_ref, w_ref, g_ref, o_ref):
    t = pl.program_id(0)

    @pl.when(t < ets_ref[1, 120])
    def _():
        hp = ra_ref[...] + rb_ref[...]
        hg = 0.5 * hp * (1.0 + jax.lax.erf(hp * 0.7071067811865476))
        acc = jax.lax.dot_general(hg.astype(jnp.bfloat16), w_ref[0],
                                  (((1,), (1,)), ((), ())),
                                  preferred_element_type=jnp.float32)
        gcol = jax.lax.transpose(g_ref[0], (1, 0))
        o_ref[...] = acc * gcol


def _gemm2(ets, ra, rb, w2b, g2s):
    g3 = g2s.reshape(NT, 1, TILE)
    grid_spec = pltpu.PrefetchScalarGridSpec(
        num_scalar_prefetch=1,
        grid=(NT,),
        in_specs=[
            pl.BlockSpec((TILE, D), lambda i, ets: (i, 0)),
            pl.BlockSpec((TILE, D), lambda i, ets: (i, 0)),
            pl.BlockSpec((1, D, D), lambda i, ets: (ets[1, i], 0, 0)),
            pl.BlockSpec((1, 1, TILE), lambda i, ets: (i, 0, 0)),
        ],
        out_specs=pl.BlockSpec((TILE, D), lambda i, ets: (i, 0)),
    )
    return pl.pallas_call(
        _gemm2_body, grid_spec=grid_spec,
        out_shape=jax.ShapeDtypeStruct((PMAX, D), jnp.float32),
    )(ets, ra, rb, w2b, g3)


def _sc_gather2_body(src, ia, ib, ra_o, rb_o, idxr, idxc, rows, sem):
    wid = lax.axis_index("s") * 2 + lax.axis_index("c")
    lo = wid * RPW
    half = RPW // 2
    for srcidx, dst in ((ia, ra_o), (ib, rb_o)):
        for c in range(2):
            pltpu.sync_copy(srcidx.at[pl.ds(lo + c * half, half)], idxr)
            _clamp_idx(idxr, idxc, half, PMAX)
            pltpu.async_copy(src.at[idxc], rows, sem).wait()
            pltpu.sync_copy(rows, dst.at[pl.ds(lo + c * half, half)])


def _sc_gather2(out1s, ia, ib):
    mesh = plsc.VectorSubcoreMesh(core_axis_name="c", subcore_axis_name="s")
    fn = pl.kernel(
        _sc_gather2_body,
        out_type=(
            jax.ShapeDtypeStruct((PMAX, D), jnp.float32),
            jax.ShapeDtypeStruct((PMAX, D), jnp.float32),
        ),
        mesh=mesh,
        scratch_types=[
            pltpu.VMEM((RPW // 2,), jnp.int32),
            pltpu.VMEM((RPW // 2,), jnp.int32),
            pltpu.VMEM((RPW // 2, D), jnp.float32),
            pltpu.SemaphoreType.DMA,
        ],
    )
    return fn(out1s, ia, ib)


def _sc_final_body(src, mi, sa_o, sb_o, idx, rows, sem):
    wid = lax.axis_index("s") * 2 + lax.axis_index("c")
    lo = wid * TPW
    for r, dst in ((2, sa_o), (3, sb_o)):
        pltpu.sync_copy(mi.at[r, pl.ds(lo, TPW)], idx)
        pltpu.async_copy(src.at[idx], rows, sem).wait()
        pltpu.sync_copy(rows, dst.at[pl.ds(lo, TPW)])


def _sc_final(out2s, mi):
    mesh = plsc.VectorSubcoreMesh(core_axis_name="c", subcore_axis_name="s")
    fn = pl.kernel(
        _sc_final_body,
        out_type=(
            jax.ShapeDtypeStruct((N, D), jnp.float32),
            jax.ShapeDtypeStruct((N, D), jnp.float32),
        ),
        mesh=mesh,
        scratch_types=[
            pltpu.VMEM((TPW,), jnp.int32),
            pltpu.VMEM((TPW, D), jnp.float32),
            pltpu.SemaphoreType.DMA,
        ],
    )
    return fn(out2s, mi)


def _final_body(sa_ref, sb_ref, bias_ref, o_ref):
    o_ref[...] = (sa_ref[...] + sb_ref[...]
                  + bias_ref[...].astype(jnp.float32))


def _final(sa, sb, bias):
    tm = 512
    return pl.pallas_call(
        _final_body,
        grid=(N // tm,),
        in_specs=[pl.BlockSpec((tm, D), lambda i: (i, 0))] * 3,
        out_specs=pl.BlockSpec((tm, D), lambda i: (i, 0)),
        out_shape=jax.ShapeDtypeStruct((N, D), jnp.float32),
    )(sa, sb, bias)


def kernel(x, P_w, U1, U2, U3, W1, W2, b2):
    Bx, Tx, Dx = x.shape
    x_flat = x.reshape(N, D)
    pwb = P_w.T.astype(jnp.bfloat16)
    ub = jnp.concatenate([U1, U2, U3], axis=0).T.astype(jnp.bfloat16)
    b2b = b2.astype(jnp.bfloat16)
    w1b = W1.astype(jnp.bfloat16)
    w2b = W2.astype(jnp.bfloat16)

    xb, bias, mi, mf, ets = _route(x_flat, pwb, ub, b2b)
    tok1, g1s, g2s, ia, ib = _sc_scatter(mi, mf)
    xs = _sc_gatherx(x_flat, tok1)
    out1s = _gemm1(ets, xs, w1b, g1s)
    ra, rb = _sc_gather2(out1s, ia, ib)
    out2s = _gemm2(ets, ra, rb, w2b, g2s)
    sa, sb = _sc_final(out2s, mi)
    y = _final(sa, sb, bias)
    return y.reshape(Bx, Tx, D)


# speedup vs baseline: 3.5129x; 3.5129x over previous
import functools

import jax
import jax.numpy as jnp
from jax.experimental import pallas as pl
from jax.experimental.pallas import tpu as pltpu

_HIGH = jax.lax.Precision.HIGHEST


def _top2_coeff(z, tau):
    idx = jax.lax.broadcasted_iota(jnp.int32, z.shape, 1)
    v1 = jnp.max(z, axis=1, keepdims=True)
    i1 = jnp.min(jnp.where(z == v1, idx, z.shape[1]), axis=1, keepdims=True)
    m1 = idx == i1
    z2 = jnp.where(m1, -jnp.inf, z)
    v2 = jnp.max(z2, axis=1, keepdims=True)
    i2 = jnp.min(jnp.where(z2 == v2, idx, z.shape[1]), axis=1, keepdims=True)
    m2 = idx == i2
    t = tau + 1e-8
    a = jnp.exp((v2 - v1) / t)
    w1 = 1.0 / (1.0 + a)
    w2 = a / (1.0 + a)
    return jnp.where(m1, w1, 0.0) + jnp.where(m2, w2, 0.0)


def _fused_body(x_ref, pw_ref, u_ref, w1_ref, w2_ref, b2_ref, out_ref):
    xt = x_ref[...]
    xb = xt.astype(jnp.bfloat16)
    addr = jax.lax.dot_general(xb, pw_ref[...], (((1,), (0,)), ((), ())),
                               preferred_element_type=jnp.float32)
    zz = jax.lax.dot_general(addr.astype(jnp.bfloat16), u_ref[...],
                             (((1,), (0,)), ((), ())),
                             preferred_element_type=jnp.float32)
    c1 = _top2_coeff(zz[:, 0:8], 1.0)
    c2 = _top2_coeff(zz[:, 8:16], 1.0)
    c3 = _top2_coeff(zz[:, 16:24], 1.0)

    h = None
    for l in range(8):
        yl = jax.lax.dot_general(xb, w1_ref[l], (((1,), (1,)), ((), ())),
                                 preferred_element_type=jnp.float32)
        h = yl * c1[:, l:l + 1] if h is None else h + yl * c1[:, l:l + 1]
    h = 0.5 * h * (1.0 + jax.lax.erf(h * 0.7071067811865476))

    hb = h.astype(jnp.bfloat16)
    y = jax.lax.dot_general(c3, b2_ref[...], (((1,), (0,)), ((), ())),
                            precision=_HIGH,
                            preferred_element_type=jnp.float32)
    for l in range(8):
        yl = jax.lax.dot_general(hb, w2_ref[l], (((1,), (1,)), ((), ())),
                                 preferred_element_type=jnp.float32)
        y = y + yl * c2[:, l:l + 1]
    out_ref[...] = y


@functools.partial(jax.jit, static_argnames=())
def kernel(x, P_w, U1, U2, U3, W1, W2, b2):
    Bx, Tx, D = x.shape
    N = Bx * Tx
    H = W1.shape[1]
    DO = W2.shape[1]
    x_flat = x.reshape(N, D)
    u_pack = jnp.concatenate([U1, U2, U3], axis=0).T.astype(jnp.bfloat16)
    pwb = P_w.T.astype(jnp.bfloat16)
    w1b = W1.astype(jnp.bfloat16)
    w2b = W2.astype(jnp.bfloat16)

    TM = 512
    grid = (N // TM,)
    out = pl.pallas_call(
        _fused_body,
        grid=grid,
        in_specs=[
            pl.BlockSpec((TM, D), lambda i: (i, 0)),
            pl.BlockSpec(pwb.shape, lambda i: (0, 0)),
            pl.BlockSpec(u_pack.shape, lambda i: (0, 0)),
            pl.BlockSpec(w1b.shape, lambda i: (0, 0, 0)),
            pl.BlockSpec(w2b.shape, lambda i: (0, 0, 0)),
            pl.BlockSpec(b2.shape, lambda i: (0, 0)),
        ],
        out_specs=pl.BlockSpec((TM, DO), lambda i: (i, 0)),
        out_shape=jax.ShapeDtypeStruct((N, DO), jnp.float32),
    )(x_flat, pwb, u_pack, w1b, w2b, b2)
    return out.reshape(Bx, Tx, DO)
